# baseline (speedup 1.0000x reference)
import functools

import jax
import jax.numpy as jnp
from jax import lax
from jax.experimental import pallas as pl
from jax.experimental.pallas import tpu as pltpu
from jax.experimental.pallas import tpu_sc as plsc

BATCH = 16384
EMBED_DIM = 32
NC = 2
NS = 16
NW = NC * NS
BPW = BATCH // NW
GROUPS = BPW // 16


def _sc_kernel(influencer_hbm, brand_hbm, inf_table_hbm, brand_table_hbm,
               wb_hbm, b16_hbm, out_hbm,
               idx_a, idx_b, rows_a, rows_b, wb_v, b_v, out_v,
               sem_a, sem_b):
    wid = lax.axis_index("s") * NC + lax.axis_index("c")
    base = wid * BPW

    pltpu.sync_copy(influencer_hbm.at[pl.ds(base, BPW)], idx_a)
    pltpu.sync_copy(brand_hbm.at[pl.ds(base, BPW)], idx_b)
    cp_a = pltpu.async_copy(inf_table_hbm.at[idx_a], rows_a, sem_a)
    cp_b = pltpu.async_copy(brand_table_hbm.at[idx_b], rows_b, sem_b)
    pltpu.sync_copy(wb_hbm, wb_v)
    pltpu.sync_copy(b16_hbm, b_v)
    cp_a.wait()
    cp_b.wait()

    lanes = lax.iota(jnp.int32, 16)

    def group_body(g, _):
        row0 = g * 16
        rows16 = row0 + lanes
        acc = b_v[:]
        for d in range(EMBED_DIM):
            col = jnp.full((16,), d, jnp.int32)
            acc = acc + plsc.load_gather(rows_a, [rows16, col]) * wb_v[d, :]
        for d in range(EMBED_DIM):
            col = jnp.full((16,), d, jnp.int32)
            acc = acc + plsc.load_gather(rows_b, [rows16, col]) * wb_v[EMBED_DIM + d, :]
        out_v[pl.ds(row0, 16)] = acc
        return 0

    lax.fori_loop(0, GROUPS, group_body, 0)
    pltpu.sync_copy(out_v, out_hbm.at[pl.ds(base, BPW)])


@jax.jit
def kernel(influencer, brand, influencer_table, brand_table, W, b):
    wb = jnp.broadcast_to(W.reshape(2 * EMBED_DIM, 1), (2 * EMBED_DIM, 16))
    b16 = jnp.broadcast_to(b, (16,))
    mesh = plsc.VectorSubcoreMesh(core_axis_name="c", subcore_axis_name="s")
    run = pl.kernel(
        _sc_kernel,
        out_type=jax.ShapeDtypeStruct((BATCH,), jnp.float32),
        mesh=mesh,
        scratch_types=[
            pltpu.VMEM((BPW,), jnp.int32),
            pltpu.VMEM((BPW,), jnp.int32),
            pltpu.VMEM((BPW, EMBED_DIM), jnp.float32),
            pltpu.VMEM((BPW, EMBED_DIM), jnp.float32),
            pltpu.VMEM((2 * EMBED_DIM, 16), jnp.float32),
            pltpu.VMEM((16,), jnp.float32),
            pltpu.VMEM((BPW,), jnp.float32),
            pltpu.SemaphoreType.DMA,
            pltpu.SemaphoreType.DMA,
        ],
        compiler_params=pltpu.CompilerParams(
            needs_layout_passes=False, use_tc_tiling_on_sc=False),
    )
    return run(influencer, brand, influencer_table, brand_table, wb, b16)


# speedup vs baseline: 1.4675x; 1.4675x over previous
import jax
import jax.numpy as jnp
from jax import lax
from jax.experimental import pallas as pl
from jax.experimental.pallas import tpu as pltpu
from jax.experimental.pallas import tpu_sc as plsc

BATCH = 16384
EMBED_DIM = 32
NC = 2
NS = 16
NW = NC * NS
BPW = BATCH // NW
HALF = BPW // 2
K = 16
NCHUNK = HALF // K
HGROUPS = HALF // 16


def _sc_kernel(influencer_hbm, brand_hbm, inf_table_hbm, brand_table_hbm,
               wb_hbm, b16_hbm, out_hbm,
               idx_as, idx_bs, rows_a, rows_b, wb_v, b_v,
               out_v, sem_a, sem_b):
    wid = lax.axis_index("s") * NC + lax.axis_index("c")
    base = wid * BPW

    pltpu.sync_copy(influencer_hbm.at[pl.ds(base, BPW)], idx_as)
    pltpu.sync_copy(brand_hbm.at[pl.ds(base, BPW)], idx_bs)

    def _lane(vec, j):
        return lax.squeeze(lax.slice(vec, (j,), (j + 1,)), (0,))
    pltpu.sync_copy(wb_hbm, wb_v)
    pltpu.sync_copy(b16_hbm, b_v)

    lanes = lax.iota(jnp.int32, 16)

    for h in range(2):
        hbase = h * HALF

        def chunk_body(c, _):
            r0 = c * K
            vec_a = idx_as[pl.ds(hbase + r0, K)]
            vec_b = idx_bs[pl.ds(hbase + r0, K)]
            for j in range(K):
                ia = _lane(vec_a, j)
                pltpu.async_copy(inf_table_hbm.at[ia], rows_a.at[r0 + j], sem_a)
            for j in range(K):
                ib = _lane(vec_b, j)
                pltpu.async_copy(brand_table_hbm.at[ib], rows_b.at[r0 + j], sem_b)
            for j in range(K):
                pltpu.make_async_copy(
                    inf_table_hbm.at[0], rows_a.at[r0 + j], sem_a).wait()
                pltpu.make_async_copy(
                    brand_table_hbm.at[0], rows_b.at[r0 + j], sem_b).wait()
            return 0

        lax.fori_loop(0, NCHUNK, chunk_body, 0)

        def group_body(g, _):
            row0 = g * 16
            rows16 = row0 + lanes
            acc = b_v[:]
            for d in range(EMBED_DIM):
                col = jnp.full((16,), d, jnp.int32)
                acc = acc + plsc.load_gather(rows_a, [rows16, col]) * wb_v[pl.ds(d * 16, 16)]
            for d in range(EMBED_DIM):
                col = jnp.full((16,), d, jnp.int32)
                acc = acc + plsc.load_gather(rows_b, [rows16, col]) * wb_v[pl.ds((EMBED_DIM + d) * 16, 16)]
            out_v[pl.ds(hbase + row0, 16)] = acc
            return 0

        lax.fori_loop(0, HGROUPS, group_body, 0)

    pltpu.sync_copy(out_v, out_hbm.at[pl.ds(base, BPW)])


@jax.jit
def kernel(influencer, brand, influencer_table, brand_table, W, b):
    wb = jnp.broadcast_to(W.reshape(2 * EMBED_DIM, 1), (2 * EMBED_DIM, 16)).reshape(-1)
    b16 = jnp.broadcast_to(b, (16,))
    mesh = plsc.VectorSubcoreMesh(core_axis_name="c", subcore_axis_name="s")
    run = pl.kernel(
        _sc_kernel,
        out_type=jax.ShapeDtypeStruct((BATCH,), jnp.float32),
        mesh=mesh,
        scratch_types=[
            pltpu.VMEM((BPW,), jnp.int32),
            pltpu.VMEM((BPW,), jnp.int32),
            pltpu.VMEM((HALF, EMBED_DIM), jnp.float32),
            pltpu.VMEM((HALF, EMBED_DIM), jnp.float32),
            pltpu.VMEM((2 * EMBED_DIM * 16,), jnp.float32),
            pltpu.VMEM((16,), jnp.float32),
            pltpu.VMEM((BPW,), jnp.float32),
            pltpu.SemaphoreType.DMA,
            pltpu.SemaphoreType.DMA,
        ],
        compiler_params=pltpu.CompilerParams(
            needs_layout_passes=False, use_tc_tiling_on_sc=True),
    )
    return run(influencer, brand, influencer_table, brand_table, wb, b16)
